# baseline (speedup 1.0000x reference)
import functools

import jax
import jax.numpy as jnp
from jax import lax
from jax.experimental import pallas as pl
from jax.experimental.pallas import tpu as pltpu
from jax.experimental.pallas import tpu_sc as plsc

N = 10000
E = 320000
D_IN = 128
H = 64
DEGW = 8
NC = 2
NS = 16
NW = NC * NS
EPW = E // NW
C = 400
NCHUNK = EPW // C
DEPTH = 2
RPT = 624
RTAIL = N - NS * RPT

_F32 = jnp.float32


def _sc_agg(compute_deg):
    mesh = plsc.VectorSubcoreMesh(core_axis_name="c", subcore_axis_name="s")
    out_type = [jax.ShapeDtypeStruct((NC, N, H), _F32)]
    scratch = (
        [pltpu.VMEM((NCHUNK, 2, C), jnp.int32)]
        + [pltpu.VMEM((C, H), _F32)] * DEPTH
        + [pltpu.VMEM_SHARED((N, H), _F32)]
        + [pltpu.SemaphoreType.DMA] * (DEPTH + 1)
    )
    if compute_deg:
        out_type.append(jax.ShapeDtypeStruct((NC, N, DEGW), _F32))
        scratch += [
            pltpu.VMEM((C, DEGW), _F32),
            pltpu.VMEM_SHARED((N, DEGW), _F32),
        ]

    def body(x_hbm, ei_hbm, z64_hbm, *rest):
        if compute_deg:
            (ones_hbm, z16_hbm, part_hbm, degpart_hbm, *mid) = rest
            *mid, ones_v, dacc_sh = mid
        else:
            (part_hbm, *mid) = rest
        idx_all = mid[0]
        rows = mid[1:1 + DEPTH]
        acc_sh = mid[1 + DEPTH]
        sem_i = mid[2 + DEPTH]
        sems = mid[3 + DEPTH:3 + 2 * DEPTH]
        bufs = tuple(zip(rows, sems))
        c = lax.axis_index("c")
        s = lax.axis_index("s")
        wid = s * NC + c
        r0 = pl.multiple_of(s * RPT, 8)
        t0 = NS * RPT
        pltpu.async_copy(ei_hbm.at[wid], idx_all, sem_i)
        pltpu.sync_copy(z64_hbm.at[pl.ds(r0, RPT)], acc_sh.at[pl.ds(r0, RPT)])
        if compute_deg:
            pltpu.sync_copy(z16_hbm.at[pl.ds(r0, RPT)], dacc_sh.at[pl.ds(r0, RPT)])
            pltpu.sync_copy(ones_hbm, ones_v)

        @pl.when(s == 0)
        def _():
            pltpu.sync_copy(z64_hbm.at[pl.ds(t0, RTAIL)],
                            acc_sh.at[pl.ds(t0, RTAIL)])
            if compute_deg:
                pltpu.sync_copy(z16_hbm.at[pl.ds(t0, RTAIL)],
                                dacc_sh.at[pl.ds(t0, RTAIL)])

        pltpu.make_async_copy(ei_hbm.at[wid], idx_all, sem_i).wait()
        plsc.subcore_barrier()

        def gstart(k, rows_v, sem):
            pltpu.async_copy(x_hbm.at[idx_all.at[k, 0]], rows_v, sem)

        def scatter(k, rows_v, sem):
            pltpu.make_async_copy(x_hbm.at[idx_all.at[k, 0]], rows_v,
                                  sem).wait()
            pltpu.sync_copy(rows_v, acc_sh.at[idx_all.at[k, 1]], add=True)
            if compute_deg:
                pltpu.sync_copy(ones_v, dacc_sh.at[idx_all.at[k, 1]],
                                add=True)

        for j in range(DEPTH):
            gstart(j, *bufs[j])

        nfull = (NCHUNK - DEPTH) // DEPTH

        def step(k, carry):
            for j in range(DEPTH):
                idx = DEPTH * k + j
                scatter(idx, *bufs[j])
                gstart(idx + DEPTH, *bufs[j])
            return carry

        lax.fori_loop(0, nfull, step, 0)
        for idx in range(DEPTH * nfull, NCHUNK):
            scatter(idx, *bufs[idx % DEPTH])
            if idx + DEPTH < NCHUNK:
                gstart(idx + DEPTH, *bufs[idx % DEPTH])
        plsc.subcore_barrier()
        pltpu.sync_copy(acc_sh.at[pl.ds(r0, RPT)],
                        part_hbm.at[c, pl.ds(r0, RPT)])
        if compute_deg:
            pltpu.sync_copy(dacc_sh.at[pl.ds(r0, RPT)],
                            degpart_hbm.at[c, pl.ds(r0, RPT)])

        @pl.when(s == 0)
        def _():
            pltpu.sync_copy(acc_sh.at[pl.ds(t0, RTAIL)],
                            part_hbm.at[c, pl.ds(t0, RTAIL)])
            if compute_deg:
                pltpu.sync_copy(dacc_sh.at[pl.ds(t0, RTAIL)],
                                degpart_hbm.at[c, pl.ds(t0, RTAIL)])

    return pl.kernel(body, out_type=out_type, mesh=mesh,
                     scratch_types=scratch,
                     compiler_params=pltpu.CompilerParams(
                         use_tc_tiling_on_sc=False))


def _sc_deg():
    mesh = plsc.VectorSubcoreMesh(core_axis_name="c", subcore_axis_name="s")
    out_type = [jax.ShapeDtypeStruct((NC, N, DEGW), _F32)]
    scratch = [
        pltpu.VMEM((NCHUNK, 2, C), jnp.int32),
        pltpu.VMEM((C, DEGW), _F32),
        pltpu.VMEM_SHARED((N, DEGW), _F32),
        pltpu.SemaphoreType.DMA,
    ]

    def body(ei_hbm, z16_hbm, ones_hbm, degpart_hbm, idx_all, ones_v,
             dacc_sh, sem_i):
        c = lax.axis_index("c")
        s = lax.axis_index("s")
        wid = s * NC + c
        r0 = pl.multiple_of(s * RPT, 8)
        t0 = NS * RPT
        pltpu.async_copy(ei_hbm.at[wid], idx_all, sem_i)
        pltpu.sync_copy(z16_hbm.at[pl.ds(r0, RPT)], dacc_sh.at[pl.ds(r0, RPT)])
        pltpu.sync_copy(ones_hbm, ones_v)

        @pl.when(s == 0)
        def _():
            pltpu.sync_copy(z16_hbm.at[pl.ds(t0, RTAIL)],
                            dacc_sh.at[pl.ds(t0, RTAIL)])

        pltpu.make_async_copy(ei_hbm.at[wid], idx_all, sem_i).wait()
        plsc.subcore_barrier()

        def step(k, carry):
            pltpu.sync_copy(ones_v, dacc_sh.at[idx_all.at[k, 1]], add=True)
            return carry

        lax.fori_loop(0, NCHUNK, step, 0)
        plsc.subcore_barrier()
        pltpu.sync_copy(dacc_sh.at[pl.ds(r0, RPT)],
                        degpart_hbm.at[c, pl.ds(r0, RPT)])

        @pl.when(s == 0)
        def _():
            pltpu.sync_copy(dacc_sh.at[pl.ds(t0, RTAIL)],
                            degpart_hbm.at[c, pl.ds(t0, RTAIL)])

    return pl.kernel(body, out_type=out_type, mesh=mesh,
                     scratch_types=scratch,
                     compiler_params=pltpu.CompilerParams(
                         use_tc_tiling_on_sc=False))


def _enc_body(x_ref, w_ref, b_ref, o_ref):
    h = lax.dot_general(x_ref[...], w_ref[...], (((1,), (1,)), ((), ())),
                        preferred_element_type=_F32)
    o_ref[...] = jnp.maximum(h + b_ref[...], 0.0)


def _comb_W(basis_ref, comp_ref):
    W = comp_ref[0, 0] * basis_ref[0]
    W = W + comp_ref[0, 1] * basis_ref[1]
    W = W + comp_ref[0, 2] * basis_ref[2]
    W = W + comp_ref[0, 3] * basis_ref[3]
    return W


def _agg_mean(p_ref, dp_ref):
    deg = dp_ref[0] + dp_ref[1]
    inv = 1.0 / jnp.maximum(deg[:, 0:1], 1.0)
    return (p_ref[0] + p_ref[1]) * inv


def _comb_body(p_ref, dp_ref, x_ref, basis_ref, comp_ref, root_ref, b_ref,
               o_ref):
    agg = _agg_mean(p_ref, dp_ref)
    W = _comb_W(basis_ref, comp_ref)
    x = x_ref[...]
    h = (lax.dot_general(agg, W, (((1,), (0,)), ((), ())),
                         preferred_element_type=_F32)
         + lax.dot_general(x, root_ref[...], (((1,), (0,)), ((), ())),
                           preferred_element_type=_F32)
         + b_ref[...])
    o_ref[...] = jnp.maximum(h, 0.0) + x


def _final_body(p_ref, dp_ref, x_ref, basis_ref, comp_ref, root_ref, b_ref,
                dw_ref, db_ref, cw_ref, cb_ref, x_out, d_out, c_out):
    _comb_body(p_ref, dp_ref, x_ref, basis_ref, comp_ref, root_ref, b_ref,
               x_out)
    x = x_out[...]
    d_out[...] = lax.dot_general(x, dw_ref[...], (((1,), (1,)), ((), ())),
                                 preferred_element_type=_F32) + db_ref[...]
    c_out[...] = lax.dot_general(x, cw_ref[...], (((1,), (1,)), ((), ())),
                                 preferred_element_type=_F32) + cb_ref[...]


_sc_agg_only = _sc_agg(False)
_sc_deg_count = _sc_deg()

_tc_enc = pl.pallas_call(
    _enc_body, out_shape=jax.ShapeDtypeStruct((N, H), _F32))
_tc_comb = pl.pallas_call(
    _comb_body, out_shape=jax.ShapeDtypeStruct((N, H), _F32))
_tc_final = pl.pallas_call(
    _final_body, out_shape=(jax.ShapeDtypeStruct((N, H), _F32),
                            jax.ShapeDtypeStruct((N, 2), _F32),
                            jax.ShapeDtypeStruct((N, 2), _F32)))


def kernel(x_node, edge_index_node, enc_W, enc_b, l0_basis, l0_comp, l0_root,
           l0_bias, l1_basis, l1_comp, l1_root, l1_bias, cls_delay_W,
           cls_delay_b, cls_cancel_W, cls_cancel_b):
    ei_chunks = jnp.transpose(
        edge_index_node.reshape(2, NW, NCHUNK, C), (1, 2, 0, 3))
    z64 = jnp.zeros((N, H), _F32)
    z16 = jnp.zeros((N, DEGW), _F32)
    ones = jnp.ones((C, DEGW), _F32)

    dp, = _sc_deg_count(ei_chunks, z16, ones)
    x0 = _tc_enc(x_node, enc_W, enc_b)
    p0, = _sc_agg_only(x0, ei_chunks, z64)
    x1 = _tc_comb(p0, dp, x0, l0_basis, l0_comp, l0_root, l0_bias)
    p1, = _sc_agg_only(x1, ei_chunks, z64)
    x2, delay_out, cancel_out = _tc_final(
        p1, dp, x1, l1_basis, l1_comp, l1_root, l1_bias,
        cls_delay_W, cls_delay_b, cls_cancel_W, cls_cancel_b)
    return (x2, delay_out, cancel_out)


# speedup vs baseline: 1.0265x; 1.0265x over previous
import functools

import jax
import jax.numpy as jnp
from jax import lax
from jax.experimental import pallas as pl
from jax.experimental.pallas import tpu as pltpu
from jax.experimental.pallas import tpu_sc as plsc

N = 10000
E = 320000
D_IN = 128
H = 64
DEGW = 8
NC = 2
NS = 16
NW = NC * NS
EPW = E // NW
C = 200
NCHUNK = EPW // C
DEPTH = 4
RPT = 624
RTAIL = N - NS * RPT

_F32 = jnp.float32


def _sc_agg(compute_deg):
    mesh = plsc.VectorSubcoreMesh(core_axis_name="c", subcore_axis_name="s")
    out_type = [jax.ShapeDtypeStruct((NC, N, H), _F32)]
    scratch = (
        [pltpu.VMEM((NCHUNK, 2, C), jnp.int32)]
        + [pltpu.VMEM((C, H), _F32)] * DEPTH
        + [pltpu.VMEM_SHARED((N, H), _F32)]
        + [pltpu.SemaphoreType.DMA] * (DEPTH + 1)
    )
    if compute_deg:
        out_type.append(jax.ShapeDtypeStruct((NC, N, DEGW), _F32))
        scratch += [
            pltpu.VMEM((C, DEGW), _F32),
            pltpu.VMEM_SHARED((N, DEGW), _F32),
        ]

    def body(x_hbm, ei_hbm, z64_hbm, *rest):
        if compute_deg:
            (ones_hbm, z16_hbm, part_hbm, degpart_hbm, *mid) = rest
            *mid, ones_v, dacc_sh = mid
        else:
            (part_hbm, *mid) = rest
        idx_all = mid[0]
        rows = mid[1:1 + DEPTH]
        acc_sh = mid[1 + DEPTH]
        sem_i = mid[2 + DEPTH]
        sems = mid[3 + DEPTH:3 + 2 * DEPTH]
        bufs = tuple(zip(rows, sems))
        c = lax.axis_index("c")
        s = lax.axis_index("s")
        wid = s * NC + c
        r0 = pl.multiple_of(s * RPT, 8)
        t0 = NS * RPT
        pltpu.async_copy(ei_hbm.at[wid], idx_all, sem_i)
        pltpu.sync_copy(z64_hbm.at[pl.ds(r0, RPT)], acc_sh.at[pl.ds(r0, RPT)])
        if compute_deg:
            pltpu.sync_copy(z16_hbm.at[pl.ds(r0, RPT)], dacc_sh.at[pl.ds(r0, RPT)])
            pltpu.sync_copy(ones_hbm, ones_v)

        @pl.when(s == 0)
        def _():
            pltpu.sync_copy(z64_hbm.at[pl.ds(t0, RTAIL)],
                            acc_sh.at[pl.ds(t0, RTAIL)])
            if compute_deg:
                pltpu.sync_copy(z16_hbm.at[pl.ds(t0, RTAIL)],
                                dacc_sh.at[pl.ds(t0, RTAIL)])

        pltpu.make_async_copy(ei_hbm.at[wid], idx_all, sem_i).wait()
        plsc.subcore_barrier()

        def gstart(k, rows_v, sem):
            pltpu.async_copy(x_hbm.at[idx_all.at[k, 0]], rows_v, sem)

        def scatter(k, rows_v, sem):
            pltpu.make_async_copy(x_hbm.at[idx_all.at[k, 0]], rows_v,
                                  sem).wait()
            pltpu.sync_copy(rows_v, acc_sh.at[idx_all.at[k, 1]], add=True)
            if compute_deg:
                pltpu.sync_copy(ones_v, dacc_sh.at[idx_all.at[k, 1]],
                                add=True)

        for j in range(DEPTH):
            gstart(j, *bufs[j])

        nfull = (NCHUNK - DEPTH) // DEPTH

        def step(k, carry):
            for j in range(DEPTH):
                idx = DEPTH * k + j
                scatter(idx, *bufs[j])
                gstart(idx + DEPTH, *bufs[j])
            return carry

        lax.fori_loop(0, nfull, step, 0)
        for idx in range(DEPTH * nfull, NCHUNK):
            scatter(idx, *bufs[idx % DEPTH])
            if idx + DEPTH < NCHUNK:
                gstart(idx + DEPTH, *bufs[idx % DEPTH])
        plsc.subcore_barrier()
        pltpu.sync_copy(acc_sh.at[pl.ds(r0, RPT)],
                        part_hbm.at[c, pl.ds(r0, RPT)])
        if compute_deg:
            pltpu.sync_copy(dacc_sh.at[pl.ds(r0, RPT)],
                            degpart_hbm.at[c, pl.ds(r0, RPT)])

        @pl.when(s == 0)
        def _():
            pltpu.sync_copy(acc_sh.at[pl.ds(t0, RTAIL)],
                            part_hbm.at[c, pl.ds(t0, RTAIL)])
            if compute_deg:
                pltpu.sync_copy(dacc_sh.at[pl.ds(t0, RTAIL)],
                                degpart_hbm.at[c, pl.ds(t0, RTAIL)])

    return pl.kernel(body, out_type=out_type, mesh=mesh,
                     scratch_types=scratch,
                     compiler_params=pltpu.CompilerParams(
                         use_tc_tiling_on_sc=False))


def _sc_deg():
    mesh = plsc.VectorSubcoreMesh(core_axis_name="c", subcore_axis_name="s")
    out_type = [jax.ShapeDtypeStruct((NC, N, DEGW), _F32)]
    scratch = [
        pltpu.VMEM((NCHUNK, 2, C), jnp.int32),
        pltpu.VMEM((C, DEGW), _F32),
        pltpu.VMEM_SHARED((N, DEGW), _F32),
        pltpu.SemaphoreType.DMA,
    ]

    def body(ei_hbm, z16_hbm, ones_hbm, degpart_hbm, idx_all, ones_v,
             dacc_sh, sem_i):
        c = lax.axis_index("c")
        s = lax.axis_index("s")
        wid = s * NC + c
        r0 = pl.multiple_of(s * RPT, 8)
        t0 = NS * RPT
        pltpu.async_copy(ei_hbm.at[wid], idx_all, sem_i)
        pltpu.sync_copy(z16_hbm.at[pl.ds(r0, RPT)], dacc_sh.at[pl.ds(r0, RPT)])
        pltpu.sync_copy(ones_hbm, ones_v)

        @pl.when(s == 0)
        def _():
            pltpu.sync_copy(z16_hbm.at[pl.ds(t0, RTAIL)],
                            dacc_sh.at[pl.ds(t0, RTAIL)])

        pltpu.make_async_copy(ei_hbm.at[wid], idx_all, sem_i).wait()
        plsc.subcore_barrier()

        def step(k, carry):
            pltpu.sync_copy(ones_v, dacc_sh.at[idx_all.at[k, 1]], add=True)
            return carry

        lax.fori_loop(0, NCHUNK, step, 0)
        plsc.subcore_barrier()
        pltpu.sync_copy(dacc_sh.at[pl.ds(r0, RPT)],
                        degpart_hbm.at[c, pl.ds(r0, RPT)])

        @pl.when(s == 0)
        def _():
            pltpu.sync_copy(dacc_sh.at[pl.ds(t0, RTAIL)],
                            degpart_hbm.at[c, pl.ds(t0, RTAIL)])

    return pl.kernel(body, out_type=out_type, mesh=mesh,
                     scratch_types=scratch,
                     compiler_params=pltpu.CompilerParams(
                         use_tc_tiling_on_sc=False))


def _enc_body(x_ref, w_ref, b_ref, o_ref):
    h = lax.dot_general(x_ref[...], w_ref[...], (((1,), (1,)), ((), ())),
                        preferred_element_type=_F32)
    o_ref[...] = jnp.maximum(h + b_ref[...], 0.0)


def _comb_W(basis_ref, comp_ref):
    W = comp_ref[0, 0] * basis_ref[0]
    W = W + comp_ref[0, 1] * basis_ref[1]
    W = W + comp_ref[0, 2] * basis_ref[2]
    W = W + comp_ref[0, 3] * basis_ref[3]
    return W


def _agg_mean(p_ref, dp_ref):
    deg = dp_ref[0] + dp_ref[1]
    inv = 1.0 / jnp.maximum(deg[:, 0:1], 1.0)
    return (p_ref[0] + p_ref[1]) * inv


def _comb_body(p_ref, dp_ref, x_ref, basis_ref, comp_ref, root_ref, b_ref,
               o_ref):
    agg = _agg_mean(p_ref, dp_ref)
    W = _comb_W(basis_ref, comp_ref)
    x = x_ref[...]
    h = (lax.dot_general(agg, W, (((1,), (0,)), ((), ())),
                         preferred_element_type=_F32)
         + lax.dot_general(x, root_ref[...], (((1,), (0,)), ((), ())),
                           preferred_element_type=_F32)
         + b_ref[...])
    o_ref[...] = jnp.maximum(h, 0.0) + x


def _final_body(p_ref, dp_ref, x_ref, basis_ref, comp_ref, root_ref, b_ref,
                dw_ref, db_ref, cw_ref, cb_ref, x_out, d_out, c_out):
    _comb_body(p_ref, dp_ref, x_ref, basis_ref, comp_ref, root_ref, b_ref,
               x_out)
    x = x_out[...]
    d_out[...] = lax.dot_general(x, dw_ref[...], (((1,), (1,)), ((), ())),
                                 preferred_element_type=_F32) + db_ref[...]
    c_out[...] = lax.dot_general(x, cw_ref[...], (((1,), (1,)), ((), ())),
                                 preferred_element_type=_F32) + cb_ref[...]


_sc_agg_only = _sc_agg(False)
_sc_deg_count = _sc_deg()

_tc_enc = pl.pallas_call(
    _enc_body, out_shape=jax.ShapeDtypeStruct((N, H), _F32))
_tc_comb = pl.pallas_call(
    _comb_body, out_shape=jax.ShapeDtypeStruct((N, H), _F32))
_tc_final = pl.pallas_call(
    _final_body, out_shape=(jax.ShapeDtypeStruct((N, H), _F32),
                            jax.ShapeDtypeStruct((N, 2), _F32),
                            jax.ShapeDtypeStruct((N, 2), _F32)))


def kernel(x_node, edge_index_node, enc_W, enc_b, l0_basis, l0_comp, l0_root,
           l0_bias, l1_basis, l1_comp, l1_root, l1_bias, cls_delay_W,
           cls_delay_b, cls_cancel_W, cls_cancel_b):
    ei_chunks = jnp.transpose(
        edge_index_node.reshape(2, NW, NCHUNK, C), (1, 2, 0, 3))
    z64 = jnp.zeros((N, H), _F32)
    z16 = jnp.zeros((N, DEGW), _F32)
    ones = jnp.ones((C, DEGW), _F32)

    dp, = _sc_deg_count(ei_chunks, z16, ones)
    x0 = _tc_enc(x_node, enc_W, enc_b)
    p0, = _sc_agg_only(x0, ei_chunks, z64)
    x1 = _tc_comb(p0, dp, x0, l0_basis, l0_comp, l0_root, l0_bias)
    p1, = _sc_agg_only(x1, ei_chunks, z64)
    x2, delay_out, cancel_out = _tc_final(
        p1, dp, x1, l1_basis, l1_comp, l1_root, l1_bias,
        cls_delay_W, cls_delay_b, cls_cancel_W, cls_cancel_b)
    return (x2, delay_out, cancel_out)
